# baseline (speedup 1.0000x reference)
import functools

import jax
import jax.numpy as jnp
from jax import lax
from jax.experimental import pallas as pl
from jax.experimental.pallas import tpu as pltpu
from jax.experimental.pallas import tpu_sc as plsc

_INFO = plsc.get_sparse_core_info()
_NC, _NS = _INFO.num_cores, _INFO.num_subcores
_NW = _NC * _NS


def _make_sc_broadcast(batch, n_rows, embed, dtype):
    rows_w = n_rows // _NW
    mesh = plsc.VectorSubcoreMesh(core_axis_name="c", subcore_axis_name="s")

    @functools.partial(
        pl.kernel,
        mesh=mesh,
        out_type=jax.ShapeDtypeStruct((batch, n_rows, embed), dtype),
        scratch_types=[
            pltpu.VMEM((rows_w, embed), dtype),
            pltpu.SemaphoreType.DMA,
        ],
    )
    def k(table_hbm, out_hbm, rows_v, sem):
        wid = lax.axis_index("s") * _NC + lax.axis_index("c")
        base = wid * rows_w
        pltpu.sync_copy(table_hbm.at[pl.ds(base, rows_w)], rows_v)
        copies = [
            pltpu.async_copy(rows_v, out_hbm.at[b].at[pl.ds(base, rows_w)], sem)
            for b in range(batch)
        ]
        for c in copies:
            c.wait()

    return k


def kernel(x, table):
    batch = x.shape[0]
    n_rows, embed = table.shape
    return _make_sc_broadcast(batch, n_rows, embed, table.dtype)(table)


# speedup vs baseline: 2.2196x; 2.2196x over previous
import jax
import jax.numpy as jnp
from jax.experimental import pallas as pl
from jax.experimental.pallas import tpu as pltpu


def _make_body(batch):
    def body(table_hbm, out_hbm, vmem, sem_in, sem_out):
        cp_in = pltpu.make_async_copy(table_hbm, vmem, sem_in)
        cp_in.start()
        cp_in.wait()
        copies = [
            pltpu.make_async_copy(vmem, out_hbm.at[b], sem_out)
            for b in range(batch)
        ]
        for c in copies:
            c.start()
        for c in copies:
            c.wait()

    return body


def kernel(x, table):
    batch = x.shape[0]
    n_rows, embed = table.shape
    return pl.pallas_call(
        _make_body(batch),
        in_specs=[pl.BlockSpec(memory_space=pl.ANY)],
        out_specs=pl.BlockSpec(memory_space=pl.ANY),
        out_shape=jax.ShapeDtypeStruct((batch, n_rows, embed), table.dtype),
        scratch_shapes=[
            pltpu.VMEM((n_rows, embed), table.dtype),
            pltpu.SemaphoreType.DMA,
            pltpu.SemaphoreType.DMA,
        ],
    )(table)


# speedup vs baseline: 2.3824x; 1.0734x over previous
import jax
import jax.numpy as jnp
from jax.experimental import pallas as pl
from jax.experimental.pallas import tpu as pltpu

_CHUNKS = 8


def _make_body(batch, n_rows):
    rows_c = n_rows // _CHUNKS

    def body(table_hbm, out_hbm, vmem, sem_rd, sem_wr):
        reads = [
            pltpu.make_async_copy(
                table_hbm.at[pl.ds(k * rows_c, rows_c)],
                vmem.at[pl.ds(k * rows_c, rows_c)],
                sem_rd.at[k],
            )
            for k in range(_CHUNKS)
        ]
        for r in reads:
            r.start()
        writes = []
        for k in range(_CHUNKS):
            reads[k].wait()
            for b in range(batch):
                w = pltpu.make_async_copy(
                    vmem.at[pl.ds(k * rows_c, rows_c)],
                    out_hbm.at[b].at[pl.ds(k * rows_c, rows_c)],
                    sem_wr,
                )
                w.start()
                writes.append(w)
        for w in writes:
            w.wait()

    return body


def kernel(x, table):
    batch = x.shape[0]
    n_rows, embed = table.shape
    return pl.pallas_call(
        _make_body(batch, n_rows),
        in_specs=[pl.BlockSpec(memory_space=pl.ANY)],
        out_specs=pl.BlockSpec(memory_space=pl.ANY),
        out_shape=jax.ShapeDtypeStruct((batch, n_rows, embed), table.dtype),
        scratch_shapes=[
            pltpu.VMEM((n_rows, embed), table.dtype),
            pltpu.SemaphoreType.DMA((_CHUNKS,)),
            pltpu.SemaphoreType.DMA,
        ],
    )(table)


# speedup vs baseline: 2.4161x; 1.0142x over previous
import jax
import jax.numpy as jnp
from jax.experimental import pallas as pl
from jax.experimental.pallas import tpu as pltpu

_CHUNKS = 16


def _make_body(batch, n_rows):
    rows_c = n_rows // _CHUNKS

    def body(table_hbm, out_hbm, vmem, sem_rd, sem_wr):
        reads = [
            pltpu.make_async_copy(
                table_hbm.at[pl.ds(k * rows_c, rows_c)],
                vmem.at[pl.ds(k * rows_c, rows_c)],
                sem_rd.at[k],
            )
            for k in range(_CHUNKS)
        ]
        for r in reads:
            r.start()
        writes = []
        for k in range(_CHUNKS):
            reads[k].wait()
            for b in range(batch):
                w = pltpu.make_async_copy(
                    vmem.at[pl.ds(k * rows_c, rows_c)],
                    out_hbm.at[b].at[pl.ds(k * rows_c, rows_c)],
                    sem_wr,
                )
                w.start()
                writes.append(w)
        for w in writes:
            w.wait()

    return body


def kernel(x, table):
    batch = x.shape[0]
    n_rows, embed = table.shape
    return pl.pallas_call(
        _make_body(batch, n_rows),
        in_specs=[pl.BlockSpec(memory_space=pl.ANY)],
        out_specs=pl.BlockSpec(memory_space=pl.ANY),
        out_shape=jax.ShapeDtypeStruct((batch, n_rows, embed), table.dtype),
        scratch_shapes=[
            pltpu.VMEM((n_rows, embed), table.dtype),
            pltpu.SemaphoreType.DMA((_CHUNKS,)),
            pltpu.SemaphoreType.DMA,
        ],
    )(table)
